# baseline (speedup 1.0000x reference)
import jax
import jax.numpy as jnp
from jax.experimental import pallas as pl
from jax.experimental.pallas import tpu as pltpu

_BM = 512
_HK = 2048


def _dot(a, b):
    return jax.lax.dot_general(
        a, b,
        dimension_numbers=(((1,), (0,)), ((), ())),
        preferred_element_type=jnp.float32,
    )


def _matmul_block(mat_ref, inp_hbm, out_ref, inp_vmem, sems):
    i = pl.program_id(0)

    @pl.when(i == 0)
    def _():
        copies = [
            pltpu.make_async_copy(
                inp_hbm.at[pl.ds(j * _HK, _HK), :],
                inp_vmem.at[pl.ds(j * _HK, _HK), :],
                sems.at[j],
            )
            for j in range(2)
        ]
        for c in copies:
            c.start()
        copies[0].wait()
        first = _dot(mat_ref[:, :_HK], inp_vmem[:_HK, :])
        copies[1].wait()
        out_ref[...] = first + _dot(mat_ref[:, _HK:], inp_vmem[_HK:, :])

    @pl.when(i != 0)
    def _():
        out_ref[...] = _dot(mat_ref[...], inp_vmem[...])


def kernel(inp, matrix):
    B, C, S = inp.shape
    M, K = matrix.shape
    inp_flat = inp.reshape(B * C, S)

    out_flat = pl.pallas_call(
        _matmul_block,
        grid=(M // _BM,),
        in_specs=[
            pl.BlockSpec((_BM, K), lambda i: (i, 0)),
            pl.BlockSpec(memory_space=pltpu.MemorySpace.HBM),
        ],
        out_specs=pl.BlockSpec((_BM, S), lambda i: (i, 0)),
        out_shape=jax.ShapeDtypeStruct((M, S), jnp.float32),
        scratch_shapes=[
            pltpu.VMEM((B * C, S), jnp.float32),
            pltpu.SemaphoreType.DMA((2,)),
        ],
        compiler_params=pltpu.CompilerParams(
            dimension_semantics=("arbitrary",),
        ),
    )(matrix, inp_flat)

    return out_flat.reshape(B, C, S)


# speedup vs baseline: 1.0495x; 1.0495x over previous
import jax
import jax.numpy as jnp
from jax.experimental import pallas as pl
from jax.experimental.pallas import tpu as pltpu

_BM = 512


def _matmul_block(mat_ref, inp_ref, out_ref):
    out_ref[...] = jax.lax.dot_general(
        mat_ref[...],
        inp_ref[...],
        dimension_numbers=(((1,), (0,)), ((), ())),
        preferred_element_type=jnp.float32,
    )


def kernel(inp, matrix):
    B, C, S = inp.shape
    M, K = matrix.shape
    inp_flat = inp.reshape(B * C, S)

    out_flat = pl.pallas_call(
        _matmul_block,
        grid=(M // _BM,),
        in_specs=[
            pl.BlockSpec((_BM, K), lambda i: (i, 0)),
            pl.BlockSpec((B * C, S), lambda i: (0, 0)),
        ],
        out_specs=pl.BlockSpec((_BM, S), lambda i: (i, 0)),
        out_shape=jax.ShapeDtypeStruct((M, S), jnp.float32),
        compiler_params=pltpu.CompilerParams(
            dimension_semantics=("arbitrary",),
        ),
    )(matrix, inp_flat)

    return out_flat.reshape(B, C, S)
